# Initial kernel scaffold; baseline (speedup 1.0000x reference)
import jax
import jax.numpy as jnp
from jax.experimental import pallas as pl


def kernel(inputs, hidden_state, edge_weight, Wg0, bg0, Wc0, bc0, Wg1, bg1, Wc1, bc1, Wp, bp, edge_index):
    raise NotImplementedError("write your pallas kernel here")


# speedup vs baseline: 24.0357x; 24.0357x over previous
import functools

import jax
import jax.numpy as jnp
from jax import lax
from jax.experimental import pallas as pl
from jax.experimental.pallas import tpu as pltpu
from jax.experimental.pallas import tpu_sc as plsc

_N = 10000
_E = 160000
_B = 8
_RU = 64
_OD = 1

_NSUB = 16
_C = 128
_NCH = 80
_EPW = _NCH * _C
_EP = _EPW * _NSUB
_NA = 10112
_STRIPE = _NA // _NSUB


_W = 128
_NK = _W // 16


def _make_prop(P, tstride):
    ppc = (P + 1) // 2
    mesh = plsc.VectorSubcoreMesh(core_axis_name="c", subcore_axis_name="s")

    @functools.partial(
        pl.kernel,
        out_type=jax.ShapeDtypeStruct((P * _NA, _W), jnp.float32),
        mesh=mesh,
        scratch_types=[
            pltpu.VMEM((_NCH, _C), jnp.int32),
            pltpu.VMEM((_NCH, _C), jnp.int32),
            pltpu.VMEM((_NCH, _C), jnp.float32),
            pltpu.VMEM((_C,), jnp.int32),
            pltpu.VMEM((_C, _W), jnp.float32),
            pltpu.VMEM_SHARED((_NA, _W), jnp.float32),
            pltpu.SemaphoreType.DMA,
        ],
    )
    def prop(tbl, srcR, dstR, ewR, zeros, out, src_vm, dst_vm, ew_vm, src2,
             rows, acc, sem):
        c = lax.axis_index("c")
        s = lax.axis_index("s")

        pltpu.sync_copy(srcR.at[pl.ds(s * _NCH, _NCH)], src_vm)
        pltpu.sync_copy(dstR.at[pl.ds(s * _NCH, _NCH)], dst_vm)
        pltpu.sync_copy(ewR.at[pl.ds(s * _NCH, _NCH)], ew_vm)

        def do_pass(i, carry):
            p = c + 2 * i

            @pl.when(p < P)
            def _():
                pltpu.sync_copy(zeros, acc.at[pl.ds(s * _STRIPE, _STRIPE)])
                plsc.subcore_barrier()
                off = p * tstride

                def chunk(ci, cc):
                    for kk in range(_C // 16):
                        src2[pl.ds(kk * 16, 16)] = (
                            src_vm[ci, pl.ds(kk * 16, 16)] + off)
                    pltpu.async_copy(tbl.at[src2], rows, sem).wait()

                    def scale(g, gg):
                        w16 = ew_vm[ci, pl.ds(g * 16, 16)]
                        for j in range(16):
                            e = g * 16 + j
                            w = w16[j]
                            for kk in range(_NK):
                                rows[e, pl.ds(kk * 16, 16)] = (
                                    rows[e, pl.ds(kk * 16, 16)] * w)
                        return gg

                    lax.fori_loop(0, _C // 16, scale, 0)
                    pltpu.sync_copy(rows, acc.at[dst_vm.at[ci]], add=True)
                    return cc

                lax.fori_loop(0, _NCH, chunk, 0)
                plsc.subcore_barrier()
                fb = s * _STRIPE
                pltpu.sync_copy(acc.at[pl.ds(fb, _STRIPE)],
                                out.at[pl.ds(p * _NA + fb, _STRIPE)])
                plsc.subcore_barrier()

            return carry

        lax.fori_loop(0, ppc, do_pass, 0)

    return prop


_prop_5n = _make_prop(5, _N)
_prop_5a = _make_prop(5, _NA)
_prop_4n = _make_prop(4, _N)
_prop_4a = _make_prop(4, _NA)
_prop_8n = _make_prop(8, _N)
_prop_8a = _make_prop(8, _NA)


_T = 2000
_G = (_B * _N) // _T
_f32 = jnp.float32
_R = _B * _N


def _row_spec(cols):
    return pl.BlockSpec((_T, cols), lambda i: (i, 0))


def _full_spec(r, c):
    return pl.BlockSpec((r, c), lambda i: (0, 0))


def _dot(a, b):
    return jnp.dot(a, b, preferred_element_type=jnp.float32)


def _a1_body(xf, h, ax, ah, a2x, a2h, wg, bg, rh_o, u_o):
    w = wg[...]
    g = (xf[...] * w[0:1] + _dot(h[...], w[1:65])
         + ax[...] * w[65:66] + _dot(ah[...], w[66:130])
         + a2x[...] * w[130:131] + _dot(a2h[...], w[131:195]) + bg[...])
    sg = jax.nn.sigmoid(g)
    rh_o[...] = sg[:, 0:_RU] * h[...]
    u_o[...] = sg[:, _RU:]


def _b1_body(xf, h, ax, a2x, rh, c1, c2, u, wc, bc, h0_o):
    w = wc[...]
    cand = jnp.tanh(
        xf[...] * w[0:1] + _dot(rh[...], w[1:65])
        + ax[...] * w[65:66] + _dot(c1[...], w[66:130])
        + a2x[...] * w[130:131] + _dot(c2[...], w[131:195]) + bc[...])
    uu = u[...]
    h0_o[...] = uu * h[...] + (1.0 - uu) * cand


def _a2_body(h0, h1, p1, p2, p3, p4, wg, bg, rh_o, u_o):
    w = wg[...]
    g = (_dot(h0[...], w[0:64]) + _dot(h1[...], w[64:128])
         + _dot(p1[...], w[128:192]) + _dot(p2[...], w[192:256])
         + _dot(p3[...], w[256:320]) + _dot(p4[...], w[320:384]) + bg[...])
    sg = jax.nn.sigmoid(g)
    rh_o[...] = sg[:, 0:_RU] * h1[...]
    u_o[...] = sg[:, _RU:]


def _b2_body(h0, h1, rh2, p1, p3, c1, c2, u, wc, bc, wp, bp, h1_o, pr_o):
    w = wc[...]
    cand = jnp.tanh(
        _dot(h0[...], w[0:64]) + _dot(rh2[...], w[64:128])
        + _dot(p1[...], w[128:192]) + _dot(c1[...], w[192:256])
        + _dot(p3[...], w[256:320]) + _dot(c2[...], w[320:384]) + bc[...])
    uu = u[...]
    h1n = uu * h1[...] + (1.0 - uu) * cand
    h1_o[...] = h1n
    pr_o[...] = _dot(h1n, wp[...]) + bp[...]


def _rows_out(cols):
    return jax.ShapeDtypeStruct((_R, cols), _f32)


_stage_a1 = pl.pallas_call(
    _a1_body, grid=(_G,),
    in_specs=[_row_spec(1), _row_spec(64), _row_spec(1), _row_spec(64),
              _row_spec(1), _row_spec(64),
              _full_spec(195, 128), _full_spec(1, 128)],
    out_specs=[_row_spec(64), _row_spec(64)],
    out_shape=[_rows_out(64), _rows_out(64)])

_stage_b1 = pl.pallas_call(
    _b1_body, grid=(_G,),
    in_specs=[_row_spec(1), _row_spec(64), _row_spec(1), _row_spec(1),
              _row_spec(64), _row_spec(64), _row_spec(64), _row_spec(64),
              _full_spec(195, 64), _full_spec(1, 64)],
    out_specs=[_row_spec(64)],
    out_shape=[_rows_out(64)])

_stage_a2 = pl.pallas_call(
    _a2_body, grid=(_G,),
    in_specs=[_row_spec(64)] * 6 + [_full_spec(384, 128), _full_spec(1, 128)],
    out_specs=[_row_spec(64), _row_spec(64)],
    out_shape=[_rows_out(64), _rows_out(64)])

_stage_b2 = pl.pallas_call(
    _b2_body, grid=(_G,),
    in_specs=[_row_spec(64)] * 8
    + [_full_spec(384, 64), _full_spec(1, 64), _full_spec(64, 1),
       _full_spec(1, 1)],
    out_specs=[_row_spec(64), _row_spec(1)],
    out_shape=[_rows_out(64), _rows_out(1)])


def _unpack2(o):
    return jnp.concatenate([o[:, :, :64], o[:, :, 64:]], axis=0).reshape(
        -1, 64)


def _pack2(hf):
    h = hf.reshape(_B, _N, 64)
    half = _B // 2
    return jnp.concatenate([h[:half], h[half:]], axis=-1).reshape(-1, _W)


def kernel(inputs, hidden_state, edge_weight, Wg0, bg0, Wc0, bc0, Wg1, bg1,
           Wc1, bc1, Wp, bp, edge_index):
    x = inputs
    h0 = hidden_state[0]
    h1 = hidden_state[1]
    src = edge_index[0]
    dst = edge_index[1]

    pad = _EP - _E
    pi = jnp.arange(pad, dtype=jnp.int32)
    srcR = jnp.concatenate([src, pi % 16]).reshape(_NSUB * _NCH, _C)
    dstR = jnp.concatenate([dst, _N + (pi % 16)]).reshape(_NSUB * _NCH, _C)
    ewR = jnp.concatenate(
        [edge_weight, jnp.zeros((pad,), _f32)]).reshape(_NSUB * _NCH, _C)
    zeros = jnp.zeros((_STRIPE, _W), _f32)
    edges = (srcR, dstR, ewR, zeros)

    xT = jnp.concatenate(
        [jnp.transpose(x[:, :, 0]), jnp.zeros((_N, _W - _B), _f32)], axis=1)
    h0f = h0.reshape(_R, _RU)
    tbl1 = jnp.concatenate([xT, _pack2(h0f)], axis=0)
    o1 = _prop_5n(tbl1, *edges)
    o2 = _prop_5a(o1, *edges)
    o1r = o1.reshape(5, _NA, _W)[:, :_N]
    o2r = o2.reshape(5, _NA, _W)[:, :_N]
    ax = jnp.transpose(o1r[0, :, :_B]).reshape(_R, 1)
    a2x = jnp.transpose(o2r[0, :, :_B]).reshape(_R, 1)
    ah = _unpack2(o1r[1:])
    a2h = _unpack2(o2r[1:])
    xf = x.reshape(_R, 1)

    rh, u = _stage_a1(xf, h0f, ax, ah, a2x, a2h, Wg0, bg0[None])

    oc1 = _prop_4n(_pack2(rh), *edges)
    oc2 = _prop_4a(oc1, *edges)
    c1 = _unpack2(oc1.reshape(4, _NA, _W)[:, :_N])
    c2 = _unpack2(oc2.reshape(4, _NA, _W)[:, :_N])
    (h0n,) = _stage_b1(xf, h0f, ax, a2x, rh, c1, c2, u, Wc0, bc0[None])

    h1f = h1.reshape(_R, _RU)
    tbl2 = jnp.concatenate(
        [h0n.reshape(_B, _N, _RU), h1], axis=-1).reshape(-1, _W)
    og1 = _prop_8n(tbl2, *edges)
    og2 = _prop_8a(og1, *edges)
    og1r = og1.reshape(_B, _NA, _W)[:, :_N]
    og2r = og2.reshape(_B, _NA, _W)[:, :_N]
    ah0 = og1r[:, :, :64].reshape(_R, _RU)
    ah1 = og1r[:, :, 64:].reshape(_R, _RU)
    a2h0 = og2r[:, :, :64].reshape(_R, _RU)
    a2h1 = og2r[:, :, 64:].reshape(_R, _RU)

    rh2, u2 = _stage_a2(h0n, h1f, ah0, ah1, a2h0, a2h1, Wg1, bg1[None])

    od1 = _prop_4n(_pack2(rh2), *edges)
    od2 = _prop_4a(od1, *edges)
    c21 = _unpack2(od1.reshape(4, _NA, _W)[:, :_N])
    c22 = _unpack2(od2.reshape(4, _NA, _W)[:, :_N])
    h1n, proj = _stage_b2(h0n, h1f, rh2, ah0, a2h0, c21, c22, u2, Wc1,
                          bc1[None], Wp, bp[None])

    out = proj.reshape(_B, _N * _OD)
    hstack = jnp.stack(
        [h0n.reshape(_B, _N, _RU), h1n.reshape(_B, _N, _RU)])
    return (out, hstack)
